# Initial kernel scaffold; baseline (speedup 1.0000x reference)
import jax
import jax.numpy as jnp
from jax.experimental import pallas as pl


def kernel(x, table, W_coor, b_coor, W_prom, b_prom):
    raise NotImplementedError("write your pallas kernel here")


# speedup vs baseline: 2.8384x; 2.8384x over previous
import functools

import jax
import jax.numpy as jnp
from jax import lax
from jax.experimental import pallas as pl
from jax.experimental.pallas import tpu as pltpu
from jax.experimental.pallas import tpu_sc as plsc

VOCAB = 4865
EMB = 8
VP = 4872
B, L_SEQ = 16384, 200
N = B * L_SEQ

NC, NS, LANES = 2, 16, 16
NW = NC * NS
PER_W = N // NW
CHUNK = 2048
N_CHUNKS = PER_W // CHUNK


def _fuse_body(tab_ref, wcT_ref, bc_ref, wpT_ref, bp_ref, outc_ref, outp_ref):
    t = tab_ref[...]
    outc_ref[...] = jnp.dot(t, wcT_ref[...], preferred_element_type=jnp.float32) + bc_ref[...]
    outp_ref[...] = jnp.dot(t, wpT_ref[...], preferred_element_type=jnp.float32) + bp_ref[...]


def _fuse_tables(table_pad, wcT, bc2, wpT, bp2):
    return pl.pallas_call(
        _fuse_body,
        out_shape=(
            jax.ShapeDtypeStruct((VP, 4), jnp.float32),
            jax.ShapeDtypeStruct((VP, 5), jnp.float32),
        ),
    )(table_pad, wcT, bc2, wpT, bp2)


@functools.partial(
    pl.kernel,
    out_type=(
        jax.ShapeDtypeStruct((N * 4,), jnp.float32),
        jax.ShapeDtypeStruct((N * 5,), jnp.float32),
    ),
    mesh=plsc.VectorSubcoreMesh(core_axis_name="c", subcore_axis_name="s"),
    compiler_params=pltpu.CompilerParams(needs_layout_passes=False),
    scratch_types=[
        pltpu.VMEM((CHUNK,), jnp.int32),
        pltpu.VMEM((VP * 4,), jnp.float32),
        pltpu.VMEM((VP * 5,), jnp.float32),
        pltpu.VMEM((CHUNK * 4,), jnp.float32),
        pltpu.VMEM((CHUNK * 5,), jnp.float32),
    ],
)
def _gather_kernel(idx_hbm, tc_hbm, tp_hbm, outc_hbm, outp_hbm,
                   idxv, tcv, tpv, coorv, promv):
    wid = lax.axis_index("s") * NC + lax.axis_index("c")
    base = wid * PER_W

    pltpu.sync_copy(tc_hbm, tcv)
    pltpu.sync_copy(tp_hbm, tpv)

    lanes = lax.iota(jnp.int32, LANES)
    p4 = lanes * 4
    p5 = lanes * 5

    def step(i, carry):
        idx = idxv[pl.ds(i * LANES, LANES)]
        i4 = idx * 4
        i5 = idx * 5
        for c in range(4):
            g = plsc.load_gather(tcv, [i4 + c])
            plsc.store_scatter(coorv, [p4 + (i * (LANES * 4) + c)], g)
        for c in range(5):
            g = plsc.load_gather(tpv, [i5 + c])
            plsc.store_scatter(promv, [p5 + (i * (LANES * 5) + c)], g)
        return carry

    def chunk(j, carry):
        cb = base + j * CHUNK
        pltpu.sync_copy(idx_hbm.at[pl.ds(cb, CHUNK)], idxv)
        lax.fori_loop(0, CHUNK // LANES, step, 0)
        pltpu.sync_copy(coorv, outc_hbm.at[pl.ds(cb * 4, CHUNK * 4)])
        pltpu.sync_copy(promv, outp_hbm.at[pl.ds(cb * 5, CHUNK * 5)])
        return carry

    lax.fori_loop(0, N_CHUNKS, chunk, 0)


def kernel(x, table, W_coor, b_coor, W_prom, b_prom):
    table_pad = jnp.zeros((VP, EMB), jnp.float32).at[:VOCAB].set(table)
    tc, tp = _fuse_tables(
        table_pad,
        W_coor.T.astype(jnp.float32),
        b_coor.reshape(1, 4).astype(jnp.float32),
        W_prom.T.astype(jnp.float32),
        b_prom.reshape(1, 5).astype(jnp.float32),
    )
    x_flat = x.reshape(-1).astype(jnp.int32)
    outc, outp = _gather_kernel(x_flat, tc.reshape(-1), tp.reshape(-1))
    return outc.reshape(B, L_SEQ, 4), outp.reshape(B, L_SEQ, 5)


# speedup vs baseline: 25.4158x; 8.9543x over previous
import functools

import jax
import jax.numpy as jnp
from jax import lax
from jax.experimental import pallas as pl
from jax.experimental.pallas import tpu as pltpu
from jax.experimental.pallas import tpu_sc as plsc

VOCAB = 4865
EMB = 8
VP = 4872
B, L_SEQ = 16384, 200

NC, NS, LANES = 2, 16, 16
NW = NC * NS
CHUNK = 2048
N_CHUNKS = B // CHUNK
L_BIG, N_BIG = 7, 8


def _fuse_body(tab_ref, wcT_ref, bc_ref, wpT_ref, bp_ref, outc_ref, outp_ref):
    t = tab_ref[...]
    outc_ref[...] = jnp.dot(t, wcT_ref[...], preferred_element_type=jnp.float32) + bc_ref[...]
    outp_ref[...] = jnp.dot(t, wpT_ref[...], preferred_element_type=jnp.float32) + bp_ref[...]


def _fuse_tables(table_pad, wcT, bc2, wpT, bp2):
    return pl.pallas_call(
        _fuse_body,
        out_shape=(
            jax.ShapeDtypeStruct((VP, 4), jnp.float32),
            jax.ShapeDtypeStruct((VP, 5), jnp.float32),
        ),
    )(table_pad, wcT, bc2, wpT, bp2)


@functools.partial(
    pl.kernel,
    out_type=(
        jax.ShapeDtypeStruct((4, L_SEQ, B), jnp.float32),
        jax.ShapeDtypeStruct((5, L_SEQ, B), jnp.float32),
    ),
    mesh=plsc.VectorSubcoreMesh(core_axis_name="c", subcore_axis_name="s"),
    compiler_params=pltpu.CompilerParams(needs_layout_passes=False),
    scratch_types=[
        pltpu.VMEM((CHUNK,), jnp.int32),
        pltpu.VMEM((VP * 4,), jnp.float32),
        pltpu.VMEM((VP * 5,), jnp.float32),
        pltpu.VMEM((4, CHUNK), jnp.float32),
        pltpu.VMEM((5, CHUNK), jnp.float32),
    ],
)
def _gather_kernel(xT_hbm, tc_hbm, tp_hbm, outc_hbm, outp_hbm,
                   idxv, tcv, tpv, coorv, promv):
    wid = lax.axis_index("s") * NC + lax.axis_index("c")
    is_big = wid < N_BIG
    l0 = jnp.where(is_big, L_BIG * wid, N_BIG * L_BIG + (L_BIG - 1) * (wid - N_BIG))
    nl = jnp.where(is_big, L_BIG, L_BIG - 1)

    pltpu.sync_copy(tc_hbm, tcv)
    pltpu.sync_copy(tp_hbm, tpv)

    def step(i, carry):
        idx = idxv[pl.ds(i * LANES, LANES)]
        i4 = idx * 4
        i5 = idx * 5
        o = i * LANES
        for c in range(4):
            coorv[c, pl.ds(o, LANES)] = plsc.load_gather(tcv, [i4 + c])
        for c in range(5):
            promv[c, pl.ds(o, LANES)] = plsc.load_gather(tpv, [i5 + c])
        return carry

    def chunk(args):
        l, j = args
        b0 = j * CHUNK
        pltpu.sync_copy(xT_hbm.at[l, pl.ds(b0, CHUNK)], idxv)
        lax.fori_loop(0, CHUNK // LANES, step, 0)
        pltpu.sync_copy(coorv, outc_hbm.at[:, l, pl.ds(b0, CHUNK)])
        pltpu.sync_copy(promv, outp_hbm.at[:, l, pl.ds(b0, CHUNK)])

    def l_loop(il, carry):
        l = l0 + il
        lax.fori_loop(0, N_CHUNKS, lambda j, c: (chunk((l, j)), c)[1], 0)
        return carry

    lax.fori_loop(0, nl, l_loop, 0)


def kernel(x, table, W_coor, b_coor, W_prom, b_prom):
    table_pad = jnp.zeros((VP, EMB), jnp.float32).at[:VOCAB].set(table)
    tc, tp = _fuse_tables(
        table_pad,
        W_coor.T.astype(jnp.float32),
        b_coor.reshape(1, 4).astype(jnp.float32),
        W_prom.T.astype(jnp.float32),
        b_prom.reshape(1, 5).astype(jnp.float32),
    )
    xT = x.T.astype(jnp.int32)
    outc_t, outp_t = _gather_kernel(xT, tc.reshape(-1), tp.reshape(-1))
    return jnp.transpose(outc_t, (2, 1, 0)), jnp.transpose(outp_t, (2, 1, 0))


# speedup vs baseline: 29.6098x; 1.1650x over previous
import functools

import jax
import jax.numpy as jnp
from jax import lax
from jax.experimental import pallas as pl
from jax.experimental.pallas import tpu as pltpu
from jax.experimental.pallas import tpu_sc as plsc

VOCAB = 4865
EMB = 8
VP = 4872
B, L_SEQ = 16384, 200

NC, NS, LANES = 2, 16, 16
NW = NC * NS
CHUNK = 2048
N_CHUNKS = B // CHUNK
L_BIG, N_BIG = 7, 8


def _fuse_body(tab_ref, wcT_ref, bc_ref, wpT_ref, bp_ref, outc_ref, outp_ref):
    t = tab_ref[...]
    outc_ref[...] = jnp.dot(t, wcT_ref[...], preferred_element_type=jnp.float32) + bc_ref[...]
    outp_ref[...] = jnp.dot(t, wpT_ref[...], preferred_element_type=jnp.float32) + bp_ref[...]


def _fuse_tables(table_pad, wcT, bc2, wpT, bp2):
    return pl.pallas_call(
        _fuse_body,
        out_shape=(
            jax.ShapeDtypeStruct((VP, 4), jnp.float32),
            jax.ShapeDtypeStruct((VP, 5), jnp.float32),
        ),
    )(table_pad, wcT, bc2, wpT, bp2)


@functools.partial(
    pl.kernel,
    out_type=(
        jax.ShapeDtypeStruct((4, L_SEQ, B), jnp.float32),
        jax.ShapeDtypeStruct((5, L_SEQ, B), jnp.float32),
    ),
    mesh=plsc.VectorSubcoreMesh(core_axis_name="c", subcore_axis_name="s"),
    compiler_params=pltpu.CompilerParams(needs_layout_passes=False),
    scratch_types=[
        pltpu.VMEM((2, CHUNK), jnp.int32),
        pltpu.VMEM((VP * 4,), jnp.float32),
        pltpu.VMEM((VP * 5,), jnp.float32),
        pltpu.VMEM((2, 4, CHUNK), jnp.float32),
        pltpu.VMEM((2, 5, CHUNK), jnp.float32),
        pltpu.SemaphoreType.DMA,
        pltpu.SemaphoreType.DMA,
        pltpu.SemaphoreType.DMA,
        pltpu.SemaphoreType.DMA,
        pltpu.SemaphoreType.DMA,
        pltpu.SemaphoreType.DMA,
    ],
)
def _gather_kernel(xT_hbm, tc_hbm, tp_hbm, outc_hbm, outp_hbm,
                   idxv, tcv, tpv, coorv, promv,
                   sin0, sin1, sco0, sco1, spo0, spo1):
    wid = lax.axis_index("s") * NC + lax.axis_index("c")
    is_big = wid < N_BIG
    l0 = jnp.where(is_big, L_BIG * wid, N_BIG * L_BIG + (L_BIG - 1) * (wid - N_BIG))
    nl = jnp.where(is_big, L_BIG, L_BIG - 1)
    units = nl * N_CHUNKS

    sin = [sin0, sin1]
    sco = [sco0, sco1]
    spo = [spo0, spo1]

    pltpu.sync_copy(tc_hbm, tcv)
    pltpu.sync_copy(tp_hbm, tpv)

    def l_of(u):
        return l0 + u // N_CHUNKS

    def b_of(u):
        return (u % N_CHUNKS) * CHUNK

    def start_in(u, p):
        pltpu.async_copy(
            xT_hbm.at[l_of(u), pl.ds(b_of(u), CHUNK)], idxv.at[p], sin[p])

    def gather_unit(p):
        def step(i, carry):
            idx = idxv[p, pl.ds(i * LANES, LANES)]
            i4 = idx * 4
            i5 = idx * 5
            o = i * LANES
            for c in range(4):
                coorv[p, c, pl.ds(o, LANES)] = plsc.load_gather(tcv, [i4 + c])
            for c in range(5):
                promv[p, c, pl.ds(o, LANES)] = plsc.load_gather(tpv, [i5 + c])
            return carry
        lax.fori_loop(0, CHUNK // LANES, step, 0, unroll=4)

    def unit(u, p):
        @pl.when(u >= 2)
        def _():
            pltpu.make_async_copy(
                coorv.at[p], outc_hbm.at[:, l_of(u), pl.ds(0, CHUNK)], sco[p]).wait()
            pltpu.make_async_copy(
                promv.at[p], outp_hbm.at[:, l_of(u), pl.ds(0, CHUNK)], spo[p]).wait()
        @pl.when(u + 1 < units)
        def _():
            start_in(u + 1, 1 - p)
        pltpu.make_async_copy(
            xT_hbm.at[l_of(u), pl.ds(b_of(u), CHUNK)], idxv.at[p], sin[p]).wait()
        gather_unit(p)
        pltpu.async_copy(
            coorv.at[p], outc_hbm.at[:, l_of(u), pl.ds(b_of(u), CHUNK)], sco[p])
        pltpu.async_copy(
            promv.at[p], outp_hbm.at[:, l_of(u), pl.ds(b_of(u), CHUNK)], spo[p])

    start_in(0, 0)

    def pair(k, carry):
        unit(2 * k, 0)
        unit(2 * k + 1, 1)
        return carry

    lax.fori_loop(0, units // 2, pair, 0)

    for p in range(2):
        pltpu.make_async_copy(
            coorv.at[p], outc_hbm.at[:, 0, pl.ds(0, CHUNK)], sco[p]).wait()
        pltpu.make_async_copy(
            promv.at[p], outp_hbm.at[:, 0, pl.ds(0, CHUNK)], spo[p]).wait()


def kernel(x, table, W_coor, b_coor, W_prom, b_prom):
    table_pad = jnp.zeros((VP, EMB), jnp.float32).at[:VOCAB].set(table)
    tc, tp = _fuse_tables(
        table_pad,
        W_coor.T.astype(jnp.float32),
        b_coor.reshape(1, 4).astype(jnp.float32),
        W_prom.T.astype(jnp.float32),
        b_prom.reshape(1, 5).astype(jnp.float32),
    )
    xT = x.T.astype(jnp.int32)
    outc_t, outp_t = _gather_kernel(xT, tc.reshape(-1), tp.reshape(-1))
    return jnp.transpose(outc_t, (2, 1, 0)), jnp.transpose(outp_t, (2, 1, 0))
